# baseline (speedup 1.0000x reference)
import jax
import jax.numpy as jnp
from jax import lax
from jax.experimental import pallas as pl
from jax.experimental.pallas import tpu as pltpu
from jax.experimental.pallas import tpu_sc as plsc

BATCH = 16384
EMBED_DIM = 64
WIDE = 2 * EMBED_DIM
CHUNK = 128
NBUF = 2

_info = plsc.get_sparse_core_info()
NUM_CORES = _info.num_cores
NUM_SUBCORES = _info.num_subcores
NUM_WORKERS = NUM_CORES * NUM_SUBCORES
ROWS_PER_WORKER = BATCH // NUM_WORKERS
SET_CHUNKS = ROWS_PER_WORKER // CHUNK
TOTAL_CHUNKS = 2 * SET_CHUNKS


def _transe_kernel(ent_hbm, rel_hbm,
                   ih_hbm, ir_hbm, it_hbm, ph_hbm, pr_hbm, pt_hbm,
                   pos_out, neg_out,
                   ih_v, ir_v, it_v, ph_v, pr_v, pt_v,
                   h_v, r_v, t_v,
                   sem0, sem1):
    wid = lax.axis_index("s") * NUM_CORES + lax.axis_index("c")
    wbase = wid * ROWS_PER_WORKER
    sems = [sem0, sem1]

    pltpu.sync_copy(ih_hbm.at[wid], ih_v)
    pltpu.sync_copy(ir_hbm.at[wid], ir_v)
    pltpu.sync_copy(it_hbm.at[wid], it_v)

    def fire(g, s):
        sem = sems[s]
        sl = pl.ds(s * CHUNK, CHUNK)
        psl = pl.ds(s * CHUNK * 16, CHUNK * 16)
        return (
            pltpu.async_copy(ent_hbm.at[ih_v.at[g]], h_v.at[sl], sem),
            pltpu.async_copy(rel_hbm.at[ir_v.at[g]], r_v.at[sl], sem),
            pltpu.async_copy(ent_hbm.at[it_v.at[g]], t_v.at[sl], sem),
            pltpu.async_copy(ph_hbm.at[wid].at[g], ph_v.at[psl], sem),
            pltpu.async_copy(pr_hbm.at[wid].at[g], pr_v.at[psl], sem),
            pltpu.async_copy(pt_hbm.at[wid].at[g], pt_v.at[psl], sem),
        )

    inflight = {}
    for g in range(NBUF):
        inflight[g] = fire(g, g % NBUF)

    for g in range(TOTAL_CHUNKS):
        s = g % NBUF
        for cp in inflight.pop(g):
            cp.wait()
        base = s * CHUNK

        def row_body(i, carry):
            j = base + i
            psl = pl.ds(j * 16, 16)
            p_h = ph_v[psl]
            p_r = pr_v[psl]
            p_t = pt_v[psl]
            for k in range(EMBED_DIM // 16):
                lok = pl.ds(k * 16, 16)
                hik = pl.ds(EMBED_DIM + k * 16, 16)
                hsel = h_v[j, lok] + (h_v[j, hik] - h_v[j, lok]) * p_h
                rsel = r_v[j, lok] + (r_v[j, hik] - r_v[j, lok]) * p_r
                tsel = t_v[j, lok] + (t_v[j, hik] - t_v[j, lok]) * p_t
                h_v[j, lok] = jnp.abs(hsel + rsel - tsel)
            return carry

        lax.fori_loop(0, CHUNK, row_body, 0, unroll=2)

        out_hbm = pos_out if g < SET_CHUNKS else neg_out
        row0 = wbase + (g % SET_CHUNKS) * CHUNK
        pltpu.sync_copy(h_v.at[pl.ds(base, CHUNK)],
                        out_hbm.at[pl.ds(row0, CHUNK)])
        if g + NBUF < TOTAL_CHUNKS:
            inflight[g + NBUF] = fire(g + NBUF, s)


@jax.jit
def kernel(positive_samples, negative_samples, entity_embedding, relation_embedding):
    par_shape = (NUM_WORKERS, TOTAL_CHUNKS, CHUNK * 16)

    def prep(col_pos, col_neg):
        c = jnp.concatenate(
            [col_pos.reshape(NUM_WORKERS, SET_CHUNKS, CHUNK),
             col_neg.reshape(NUM_WORKERS, SET_CHUNKS, CHUNK)], axis=1)
        idx = c >> 1
        par = jnp.broadcast_to(
            (c & 1).astype(jnp.float32)[..., None],
            (NUM_WORKERS, TOTAL_CHUNKS, CHUNK, 16)).reshape(par_shape)
        return idx, par

    ih, ph = prep(positive_samples[:, 0], negative_samples[:, 0])
    ir, pr = prep(positive_samples[:, 1], negative_samples[:, 1])
    it, pt = prep(positive_samples[:, 2], negative_samples[:, 2])

    ent2 = entity_embedding.reshape(500000, WIDE)
    rel2 = relation_embedding.reshape(500000, WIDE)

    mesh = plsc.VectorSubcoreMesh(core_axis_name="c", subcore_axis_name="s")
    out_t = jax.ShapeDtypeStruct((BATCH, WIDE), jnp.float32)
    run = pl.kernel(
        _transe_kernel,
        out_type=(out_t, out_t),
        mesh=mesh,
        compiler_params=pltpu.CompilerParams(use_tc_tiling_on_sc=True),
        scratch_types=[
            pltpu.VMEM((TOTAL_CHUNKS, CHUNK), jnp.int32),
            pltpu.VMEM((TOTAL_CHUNKS, CHUNK), jnp.int32),
            pltpu.VMEM((TOTAL_CHUNKS, CHUNK), jnp.int32),
            pltpu.VMEM((NBUF * CHUNK * 16,), jnp.float32),
            pltpu.VMEM((NBUF * CHUNK * 16,), jnp.float32),
            pltpu.VMEM((NBUF * CHUNK * 16,), jnp.float32),
            pltpu.VMEM((NBUF * CHUNK, WIDE), jnp.float32),
            pltpu.VMEM((NBUF * CHUNK, WIDE), jnp.float32),
            pltpu.VMEM((NBUF * CHUNK, WIDE), jnp.float32),
            pltpu.SemaphoreType.DMA,
            pltpu.SemaphoreType.DMA,
        ],
    )
    pos_out, neg_out = run(ent2, rel2, ih, ir, it, ph, pr, pt)
    return pos_out[:, :EMBED_DIM], neg_out[:, :EMBED_DIM]


# speedup vs baseline: 1.0136x; 1.0136x over previous
import jax
import jax.numpy as jnp
from jax import lax
from jax.experimental import pallas as pl
from jax.experimental.pallas import tpu as pltpu
from jax.experimental.pallas import tpu_sc as plsc

BATCH = 16384
EMBED_DIM = 64
CHUNK = 128
NBUF = 4

_info = plsc.get_sparse_core_info()
NUM_CORES = _info.num_cores
NUM_SUBCORES = _info.num_subcores
NUM_WORKERS = NUM_CORES * NUM_SUBCORES
ROWS_PER_WORKER = BATCH // NUM_WORKERS
SET_CHUNKS = ROWS_PER_WORKER // CHUNK
TOTAL_CHUNKS = 2 * SET_CHUNKS


def _transe_kernel(ent_hbm, rel_hbm,
                   ph_hbm, pr_hbm, pt_hbm, nh_hbm, nr_hbm, nt_hbm,
                   pos_out, neg_out,
                   ih_v, ir_v, it_v, h_v, r_v, t_v,
                   sem0, sem1, sem2, sem3):
    wid = lax.axis_index("s") * NUM_CORES + lax.axis_index("c")
    wbase = wid * ROWS_PER_WORKER
    sems = [sem0, sem1, sem2, sem3]

    pltpu.sync_copy(ph_hbm.at[wid], ih_v.at[pl.ds(0, SET_CHUNKS)])
    pltpu.sync_copy(pr_hbm.at[wid], ir_v.at[pl.ds(0, SET_CHUNKS)])
    pltpu.sync_copy(pt_hbm.at[wid], it_v.at[pl.ds(0, SET_CHUNKS)])
    pltpu.sync_copy(nh_hbm.at[wid], ih_v.at[pl.ds(SET_CHUNKS, SET_CHUNKS)])
    pltpu.sync_copy(nr_hbm.at[wid], ir_v.at[pl.ds(SET_CHUNKS, SET_CHUNKS)])
    pltpu.sync_copy(nt_hbm.at[wid], it_v.at[pl.ds(SET_CHUNKS, SET_CHUNKS)])

    def fire(g, s):
        sem = sems[s]
        sl = pl.ds(s * CHUNK, CHUNK)
        return (
            pltpu.async_copy(ent_hbm.at[ih_v.at[g]], h_v.at[sl], sem),
            pltpu.async_copy(rel_hbm.at[ir_v.at[g]], r_v.at[sl], sem),
            pltpu.async_copy(ent_hbm.at[it_v.at[g]], t_v.at[sl], sem),
        )

    inflight = {}
    for g in range(NBUF):
        inflight[g] = fire(g, g % NBUF)

    for g in range(TOTAL_CHUNKS):
        s = g % NBUF
        for cp in inflight.pop(g):
            cp.wait()
        base = s * CHUNK

        def row_body(i, carry):
            for k in range(EMBED_DIM // 16):
                sl = pl.ds(k * 16, 16)
                h_v[base + i, sl] = jnp.abs(
                    h_v[base + i, sl] + r_v[base + i, sl] - t_v[base + i, sl])
            return carry

        lax.fori_loop(0, CHUNK, row_body, 0, unroll=4)

        out_hbm = pos_out if g < SET_CHUNKS else neg_out
        row0 = wbase + (g % SET_CHUNKS) * CHUNK
        pltpu.sync_copy(h_v.at[pl.ds(base, CHUNK)],
                        out_hbm.at[pl.ds(row0, CHUNK)])
        if g + NBUF < TOTAL_CHUNKS:
            inflight[g + NBUF] = fire(g + NBUF, s)


@jax.jit
def kernel(positive_samples, negative_samples, entity_embedding, relation_embedding):
    idx_shape = (NUM_WORKERS, SET_CHUNKS, CHUNK)
    ph = positive_samples[:, 0].reshape(idx_shape)
    pr = positive_samples[:, 1].reshape(idx_shape)
    pt = positive_samples[:, 2].reshape(idx_shape)
    nh = negative_samples[:, 0].reshape(idx_shape)
    nr = negative_samples[:, 1].reshape(idx_shape)
    nt = negative_samples[:, 2].reshape(idx_shape)

    mesh = plsc.VectorSubcoreMesh(core_axis_name="c", subcore_axis_name="s")
    out_t = jax.ShapeDtypeStruct((BATCH, EMBED_DIM), jnp.float32)
    run = pl.kernel(
        _transe_kernel,
        out_type=(out_t, out_t),
        mesh=mesh,
        compiler_params=pltpu.CompilerParams(use_tc_tiling_on_sc=False),
        scratch_types=[
            pltpu.VMEM((TOTAL_CHUNKS, CHUNK), jnp.int32),
            pltpu.VMEM((TOTAL_CHUNKS, CHUNK), jnp.int32),
            pltpu.VMEM((TOTAL_CHUNKS, CHUNK), jnp.int32),
            pltpu.VMEM((NBUF * CHUNK, EMBED_DIM), jnp.float32),
            pltpu.VMEM((NBUF * CHUNK, EMBED_DIM), jnp.float32),
            pltpu.VMEM((NBUF * CHUNK, EMBED_DIM), jnp.float32),
            pltpu.SemaphoreType.DMA,
            pltpu.SemaphoreType.DMA,
            pltpu.SemaphoreType.DMA,
            pltpu.SemaphoreType.DMA,
        ],
    )
    pos_out, neg_out = run(entity_embedding, relation_embedding,
                           ph, pr, pt, nh, nr, nt)
    return pos_out, neg_out
